# baseline (speedup 1.0000x reference)
import functools

import jax
import jax.numpy as jnp
from jax import lax
from jax.experimental import pallas as pl
from jax.experimental.pallas import tpu as pltpu
from jax.experimental.pallas import tpu_sc as plsc

EMB = 64
K = 128
M = 4
NBUF = 2
H = 1


@functools.lru_cache(maxsize=None)
def _build(n_rows: int, emb: int):
    info = plsc.get_sparse_core_info()
    nc, ns = info.num_cores, info.num_subcores
    nw = nc * ns
    assert n_rows % (nw * K * M) == 0
    n = n_rows // (nw * K)
    ns_w = n // M
    assert ns_w % NBUF == 0

    mesh = plsc.VectorSubcoreMesh(core_axis_name="c", subcore_axis_name="s")
    out_t = jax.ShapeDtypeStruct((n_rows // K, K, emb), jnp.float32)

    @functools.partial(
        pl.kernel,
        mesh=mesh,
        out_type=(out_t, out_t),
        scratch_types=(
            pltpu.VMEM((n, K), jnp.int32),
            tuple(pltpu.VMEM((M, K, emb), jnp.float32) for _ in range(NBUF)),
            tuple(pltpu.SemaphoreType.DMA for _ in range(NBUF)),
            tuple(pltpu.SemaphoreType.DMA for _ in range(NBUF)),
        ),
        compiler_params=pltpu.CompilerParams(use_tc_tiling_on_sc=False),
    )
    def gather2(table_hbm, idx_a_hbm, idx_b_hbm, out_a_hbm, out_b_hbm,
                idx_v, bufs, gsems, wsems):
        wid = lax.axis_index("s") * nc + lax.axis_index("c")
        chunk_base = wid * n

        for idx_hbm, out_hbm in ((idx_a_hbm, out_a_hbm), (idx_b_hbm, out_b_hbm)):
            pltpu.sync_copy(idx_hbm.at[pl.ds(chunk_base, n), :], idx_v)
            for b in range(H):
                for c in range(M):
                    pltpu.async_copy(
                        table_hbm.at[idx_v.at[b * M + c]], bufs[b].at[c],
                        gsems[b])

            def outer(g, _):
                for b in range(NBUF):
                    j = g * NBUF + b
                    bi = (b + H) % NBUF

                    @pl.when(j + H < ns_w)
                    def _():
                        @pl.when(j + H >= NBUF)
                        def _():
                            pltpu.make_async_copy(
                                bufs[bi],
                                out_hbm.at[pl.ds(chunk_base + j * M, M)],
                                wsems[bi]).wait()
                        for c in range(M):
                            pltpu.async_copy(
                                table_hbm.at[idx_v.at[(j + H) * M + c]],
                                bufs[bi].at[c], gsems[bi])

                    for c in range(M):
                        pltpu.make_async_copy(
                            table_hbm.at[idx_v.at[j * M + c]],
                            bufs[b].at[c], gsems[b]).wait()
                    pltpu.async_copy(
                        bufs[b],
                        out_hbm.at[pl.ds(chunk_base + j * M, M)],
                        wsems[b])
                return 0

            lax.fori_loop(0, ns_w // NBUF, outer, 0)
            for b in range(NBUF):
                pltpu.make_async_copy(
                    bufs[b], out_hbm.at[pl.ds(chunk_base, M)],
                    wsems[b]).wait()

    return gather2


def kernel(table, inputs, support):
    bsz, seq = inputs.shape
    n_rows = bsz * seq
    idx_a = inputs.astype(jnp.int32).reshape(n_rows // K, K)
    idx_b = support.astype(jnp.int32).reshape(n_rows // K, K)
    fn = _build(n_rows, table.shape[1])
    out_a, out_b = fn(table, idx_a, idx_b)
    return (out_a.reshape(bsz, seq, table.shape[1]),
            out_b.reshape(bsz, seq, table.shape[1]))
